# baseline (speedup 1.0000x reference)
import functools

import jax
import jax.numpy as jnp
from jax import lax
from jax.experimental import pallas as pl
from jax.experimental.pallas import tpu as pltpu
from jax.experimental.pallas import tpu_sc as plsc

B = 4096
NC = 2
NS = 16
NW = NC * NS
BPW = B // NW
L = 16
CH = BPW // L
DB = 10


def _body(idx_hbm, betas_hbm, go_hbm, bp_hbm, tr_hbm,
          out_b, out_go, out_bp, out_tr,
          idx_v, bet_v, b_rows, go_rows, bp_rows, tr_rows,
          sem_g, sem_p, sem_t, osem):
    wid = lax.axis_index("s") * NC + lax.axis_index("c")
    base = wid * BPW

    pltpu.sync_copy(idx_hbm.at[pl.ds(base, BPW)], idx_v)
    pltpu.sync_copy(betas_hbm, bet_v)

    def chunk(c, _):
        iv = idx_v[pl.ds(c * L, L)]
        for l in range(L):
            b = c * L + l
            r = iv[l]
            pltpu.async_copy(go_hbm.at[pl.ds(r, 1)],
                             go_rows.at[pl.ds(b, 1)], sem_g)
            pltpu.async_copy(bp_hbm.at[pl.ds(r, 1)],
                             bp_rows.at[pl.ds(b, 1)], sem_p)
            pltpu.async_copy(tr_hbm.at[pl.ds(r, 1)],
                             tr_rows.at[pl.ds(b, 1)], sem_t)
        return _

    lax.fori_loop(0, CH, chunk, None)

    iota = lax.iota(jnp.int32, L)
    zeros = jnp.zeros((L,), jnp.int32)
    for m in range(5):
        lane = iota + 16 * m
        col = lane
        rsub = zeros
        for t in (10, 20, 30, 40, 50, 60, 70):
            col = jnp.where(lane >= t, lane - t, col)
            rsub = rsub + jnp.where(lane >= t, 1, 0)
        vm = plsc.load_gather(bet_v, [zeros, col])
        for r in range(BPW * DB // 80):
            plsc.store_scatter(b_rows, [8 * r + rsub, col], vm)

    pltpu.make_async_copy(go_hbm.at[pl.ds(0, BPW)], go_rows, sem_g).wait()
    pltpu.make_async_copy(bp_hbm.at[pl.ds(0, BPW)], bp_rows, sem_p).wait()
    pltpu.make_async_copy(tr_hbm.at[pl.ds(0, BPW)], tr_rows, sem_t).wait()

    ocps = [
        pltpu.async_copy(go_rows, out_go.at[pl.ds(base, BPW)], osem),
        pltpu.async_copy(tr_rows, out_tr.at[pl.ds(base, BPW)], osem),
        pltpu.async_copy(bp_rows, out_bp.at[pl.ds(base, BPW)], osem),
        pltpu.async_copy(b_rows, out_b.at[pl.ds(base, BPW)], osem),
    ]
    for cp in ocps:
        cp.wait()


def kernel(idx, betas, global_orient, body_pose, transl):
    idx = idx.astype(jnp.int32)
    dg = global_orient.shape[1]
    dp = body_pose.shape[1]
    dt = transl.shape[1]
    mesh = plsc.VectorSubcoreMesh(core_axis_name="c", subcore_axis_name="s")
    run = functools.partial(
        pl.kernel,
        mesh=mesh,
        compiler_params=pltpu.CompilerParams(needs_layout_passes=False),
        out_type=[
            jax.ShapeDtypeStruct((B, DB), jnp.float32),
            jax.ShapeDtypeStruct((B, dg), jnp.float32),
            jax.ShapeDtypeStruct((B, dp), jnp.float32),
            jax.ShapeDtypeStruct((B, dt), jnp.float32),
        ],
        scratch_types=[
            pltpu.VMEM((BPW,), jnp.int32),
            pltpu.VMEM((1, DB), jnp.float32),
            pltpu.VMEM((BPW, DB), jnp.float32),
            pltpu.VMEM((BPW, dg), jnp.float32),
            pltpu.VMEM((BPW, dp), jnp.float32),
            pltpu.VMEM((BPW, dt), jnp.float32),
            pltpu.SemaphoreType.DMA,
            pltpu.SemaphoreType.DMA,
            pltpu.SemaphoreType.DMA,
            pltpu.SemaphoreType.DMA,
        ],
    )(_body)
    ob, ogo, obp, otr = run(idx, betas, global_orient, body_pose, transl)
    return (ob, ogo, obp, otr)


# speedup vs baseline: 1.0812x; 1.0812x over previous
import functools

import jax
import jax.numpy as jnp
from jax import lax
from jax.experimental import pallas as pl
from jax.experimental.pallas import tpu as pltpu
from jax.experimental.pallas import tpu_sc as plsc

B = 4096
NC = 2
NS = 16
NW = NC * NS
BPW = B // NW
L = 16
CH = BPW // L
DB = 10


def _worker_base():
    wid = lax.axis_index("s") * NC + lax.axis_index("c")
    return wid * BPW


def _row_dmas(idx_v, srcs_dsts_sems):
    def chunk(c, _):
        iv = idx_v[pl.ds(c * L, L)]
        for l in range(L):
            b = c * L + l
            r = iv[l]
            for src, dst, sem in srcs_dsts_sems:
                pltpu.async_copy(src.at[pl.ds(r, 1)],
                                 dst.at[pl.ds(b, 1)], sem)
        return _
    lax.fori_loop(0, CH, chunk, None)


def _body_narrow(idx_hbm, betas_hbm, go_hbm, tr_hbm,
                 out_b, out_go, out_tr,
                 idx_v, bet_v, b_rows, go_rows, tr_rows,
                 sem_g, sem_t, osem):
    base = _worker_base()
    pltpu.sync_copy(idx_hbm.at[pl.ds(base, BPW)], idx_v)
    pltpu.sync_copy(betas_hbm, bet_v)

    _row_dmas(idx_v, [(go_hbm, go_rows, sem_g), (tr_hbm, tr_rows, sem_t)])

    iota = lax.iota(jnp.int32, L)
    zeros = jnp.zeros((L,), jnp.int32)
    for m in range(5):
        lane = iota + 16 * m
        col = lane
        rsub = zeros
        for t in (10, 20, 30, 40, 50, 60, 70):
            col = jnp.where(lane >= t, lane - t, col)
            rsub = rsub + jnp.where(lane >= t, 1, 0)
        vm = plsc.load_gather(bet_v, [zeros, col])
        for r in range(BPW * DB // 80):
            plsc.store_scatter(b_rows, [8 * r + rsub, col], vm)

    pltpu.make_async_copy(go_hbm.at[pl.ds(0, BPW)], go_rows, sem_g).wait()
    pltpu.make_async_copy(tr_hbm.at[pl.ds(0, BPW)], tr_rows, sem_t).wait()

    ocps = [
        pltpu.async_copy(go_rows, out_go.at[pl.ds(base, BPW)], osem),
        pltpu.async_copy(tr_rows, out_tr.at[pl.ds(base, BPW)], osem),
        pltpu.async_copy(b_rows, out_b.at[pl.ds(base, BPW)], osem),
    ]
    for cp in ocps:
        cp.wait()


def _body_wide(idx_hbm, bp_hbm, out_bp, idx_v, bp_rows, sem_p, osem):
    base = _worker_base()
    pltpu.sync_copy(idx_hbm.at[pl.ds(base, BPW)], idx_v)
    _row_dmas(idx_v, [(bp_hbm, bp_rows, sem_p)])
    pltpu.make_async_copy(bp_hbm.at[pl.ds(0, BPW)], bp_rows, sem_p).wait()
    pltpu.async_copy(bp_rows, out_bp.at[pl.ds(base, BPW)], osem).wait()


def kernel(idx, betas, global_orient, body_pose, transl):
    idx = idx.astype(jnp.int32)
    dg = global_orient.shape[1]
    dp = body_pose.shape[1]
    dt = transl.shape[1]
    mesh = plsc.VectorSubcoreMesh(core_axis_name="c", subcore_axis_name="s")
    cp = pltpu.CompilerParams(needs_layout_passes=False)

    run_narrow = functools.partial(
        pl.kernel,
        mesh=mesh,
        compiler_params=cp,
        out_type=[
            jax.ShapeDtypeStruct((B, DB), jnp.float32),
            jax.ShapeDtypeStruct((B, dg), jnp.float32),
            jax.ShapeDtypeStruct((B, dt), jnp.float32),
        ],
        scratch_types=[
            pltpu.VMEM((BPW,), jnp.int32),
            pltpu.VMEM((1, DB), jnp.float32),
            pltpu.VMEM((BPW, DB), jnp.float32),
            pltpu.VMEM((BPW, dg), jnp.float32),
            pltpu.VMEM((BPW, dt), jnp.float32),
            pltpu.SemaphoreType.DMA,
            pltpu.SemaphoreType.DMA,
            pltpu.SemaphoreType.DMA,
        ],
    )(_body_narrow)

    run_wide = functools.partial(
        pl.kernel,
        mesh=mesh,
        compiler_params=cp,
        out_type=[jax.ShapeDtypeStruct((B, dp), jnp.float32)],
        scratch_types=[
            pltpu.VMEM((BPW,), jnp.int32),
            pltpu.VMEM((BPW, dp), jnp.float32),
            pltpu.SemaphoreType.DMA,
            pltpu.SemaphoreType.DMA,
        ],
    )(_body_wide)

    ob, ogo, otr = run_narrow(idx, betas, global_orient, transl)
    (obp,) = run_wide(idx, body_pose)
    return (ob, ogo, obp, otr)


# speedup vs baseline: 1.3145x; 1.2158x over previous
import functools

import jax
import jax.numpy as jnp
from jax import lax
from jax.experimental import pallas as pl
from jax.experimental.pallas import tpu as pltpu
from jax.experimental.pallas import tpu_sc as plsc

B = 4096
NC = 2
NS = 16
NW = NC * NS
BPW = B // NW
L = 16
CH = BPW // L
DB = 10
DG = 3
DP = 69
DT = 3
N = 100000


def _body(idx_hbm, betas_hbm, go_hbm, bp_hbm, tr_hbm,
          out_b, out_go, out_bp, out_tr,
          idx_v, ego, ebp, bet_v, b_rows, go_rows, bp_rows, tr_rows,
          sem, osem):
    wid = lax.axis_index("s") * NC + lax.axis_index("c")
    base = wid * BPW

    pltpu.sync_copy(idx_hbm.at[pl.ds(base, BPW)], idx_v)
    pltpu.sync_copy(betas_hbm, bet_v)

    iota = lax.iota(jnp.int32, L)
    zeros = jnp.zeros((L,), jnp.int32)
    jconst = [(16 * k + iota) * N for k in range(5)]

    def row(b, _):
        ivec = plsc.load_gather(idx_v, [jnp.full((L,), b, jnp.int32)])
        plsc.store_scatter(ego, [DG * b + iota], jconst[0] + ivec)
        for k in range(5):
            plsc.store_scatter(ebp, [DP * b + 16 * k + iota],
                               jconst[k] + ivec)
        return _

    lax.fori_loop(0, BPW, row, None)

    cps = [
        pltpu.async_copy(go_hbm.at[ego.at[pl.ds(0, BPW * DG)]], go_rows, sem),
        pltpu.async_copy(tr_hbm.at[ego.at[pl.ds(0, BPW * DG)]], tr_rows, sem),
        pltpu.async_copy(bp_hbm.at[ebp.at[pl.ds(0, BPW * DP)]], bp_rows, sem),
    ]

    for m in range(5):
        lane = iota + 16 * m
        col = lane
        rsub = zeros
        for t in (10, 20, 30, 40, 50, 60, 70):
            col = jnp.where(lane >= t, lane - t, col)
            rsub = rsub + jnp.where(lane >= t, 1, 0)
        vm = plsc.load_gather(bet_v, [zeros, col])
        for r in range(BPW * DB // 80):
            plsc.store_scatter(b_rows, [8 * r + rsub, col], vm)

    for cp in cps:
        cp.wait()

    ocps = [
        pltpu.async_copy(go_rows, out_go.at[pl.ds(base * DG, BPW * DG)], osem),
        pltpu.async_copy(tr_rows, out_tr.at[pl.ds(base * DT, BPW * DT)], osem),
        pltpu.async_copy(bp_rows, out_bp.at[pl.ds(base * DP, BPW * DP)], osem),
        pltpu.async_copy(b_rows, out_b.at[pl.ds(base, BPW)], osem),
    ]
    for cp in ocps:
        cp.wait()


def kernel(idx, betas, global_orient, body_pose, transl):
    idx = idx.astype(jnp.int32)
    go_f = global_orient.T.reshape(-1)
    bp_f = body_pose.T.reshape(-1)
    tr_f = transl.T.reshape(-1)
    mesh = plsc.VectorSubcoreMesh(core_axis_name="c", subcore_axis_name="s")
    run = functools.partial(
        pl.kernel,
        mesh=mesh,
        compiler_params=pltpu.CompilerParams(needs_layout_passes=False),
        out_type=[
            jax.ShapeDtypeStruct((B, DB), jnp.float32),
            jax.ShapeDtypeStruct((B * DG,), jnp.float32),
            jax.ShapeDtypeStruct((B * DP,), jnp.float32),
            jax.ShapeDtypeStruct((B * DT,), jnp.float32),
        ],
        scratch_types=[
            pltpu.VMEM((BPW,), jnp.int32),
            pltpu.VMEM((BPW * DG + 16,), jnp.int32),
            pltpu.VMEM((BPW * DP + 16,), jnp.int32),
            pltpu.VMEM((1, DB), jnp.float32),
            pltpu.VMEM((BPW, DB), jnp.float32),
            pltpu.VMEM((BPW * DG,), jnp.float32),
            pltpu.VMEM((BPW * DP,), jnp.float32),
            pltpu.VMEM((BPW * DT,), jnp.float32),
            pltpu.SemaphoreType.DMA,
            pltpu.SemaphoreType.DMA,
        ],
    )(_body)
    ob, ogo, obp, otr = run(idx, betas, go_f, bp_f, tr_f)
    return (ob, ogo.reshape(B, DG), obp.reshape(B, DP), otr.reshape(B, DT))


# speedup vs baseline: 1.4478x; 1.1014x over previous
import functools

import jax
import jax.numpy as jnp
from jax import lax
from jax.experimental import pallas as pl
from jax.experimental.pallas import tpu as pltpu
from jax.experimental.pallas import tpu_sc as plsc

B = 4096
NC = 2
NS = 16
NW = NC * NS
BPW = B // NW
L = 16
DB = 10
DG = 3
DP = 69
DT = 3
N = 100000


def _worker_base():
    wid = lax.axis_index("s") * NC + lax.axis_index("c")
    return wid * BPW


def _body_narrow(idx_hbm, betas_hbm, go_hbm, tr_hbm,
                 out_b, out_go, out_tr,
                 idx_v, ego, bet_v, b_rows, go_rows, tr_rows,
                 sem, osem):
    base = _worker_base()
    pltpu.sync_copy(idx_hbm.at[pl.ds(base, BPW)], idx_v)
    pltpu.sync_copy(betas_hbm, bet_v)

    iota = lax.iota(jnp.int32, L)
    zeros = jnp.zeros((L,), jnp.int32)
    jconst0 = iota * N

    def row(b, _):
        ivec = plsc.load_gather(idx_v, [jnp.full((L,), b, jnp.int32)])
        plsc.store_scatter(ego, [DG * b + iota], jconst0 + ivec)
        return _

    lax.fori_loop(0, BPW, row, None)

    cps = [
        pltpu.async_copy(go_hbm.at[ego.at[pl.ds(0, BPW * DG)]], go_rows, sem),
        pltpu.async_copy(tr_hbm.at[ego.at[pl.ds(0, BPW * DG)]], tr_rows, sem),
    ]

    for m in range(5):
        lane = iota + 16 * m
        col = lane
        rsub = zeros
        for t in (10, 20, 30, 40, 50, 60, 70):
            col = jnp.where(lane >= t, lane - t, col)
            rsub = rsub + jnp.where(lane >= t, 1, 0)
        vm = plsc.load_gather(bet_v, [zeros, col])
        for r in range(BPW * DB // 80):
            plsc.store_scatter(b_rows, [8 * r + rsub, col], vm)

    for cp in cps:
        cp.wait()

    ocps = [
        pltpu.async_copy(go_rows, out_go.at[pl.ds(base * DG, BPW * DG)], osem),
        pltpu.async_copy(tr_rows, out_tr.at[pl.ds(base * DT, BPW * DT)], osem),
        pltpu.async_copy(b_rows, out_b.at[pl.ds(base, BPW)], osem),
    ]
    for cp in ocps:
        cp.wait()


def _body_wide(idx_hbm, bp_hbm, out_bp, idx_v, ebp, bp_rows, sem, osem):
    base = _worker_base()
    pltpu.sync_copy(idx_hbm.at[pl.ds(base, BPW)], idx_v)

    iota = lax.iota(jnp.int32, L)
    jconst = [(16 * k + iota) * N for k in range(5)]

    def row(b, _):
        ivec = plsc.load_gather(idx_v, [jnp.full((L,), b, jnp.int32)])
        for k in range(5):
            plsc.store_scatter(ebp, [DP * b + 16 * k + iota],
                               jconst[k] + ivec)
        return _

    lax.fori_loop(0, BPW, row, None)

    pltpu.async_copy(bp_hbm.at[ebp.at[pl.ds(0, BPW * DP)]],
                     bp_rows, sem).wait()
    pltpu.async_copy(bp_rows, out_bp.at[pl.ds(base * DP, BPW * DP)],
                     osem).wait()


def kernel(idx, betas, global_orient, body_pose, transl):
    idx = idx.astype(jnp.int32)
    go_f = global_orient.T.reshape(-1)
    bp_f = body_pose.T.reshape(-1)
    tr_f = transl.T.reshape(-1)
    mesh = plsc.VectorSubcoreMesh(core_axis_name="c", subcore_axis_name="s")
    cp = pltpu.CompilerParams(needs_layout_passes=False)

    run_narrow = functools.partial(
        pl.kernel,
        mesh=mesh,
        compiler_params=cp,
        out_type=[
            jax.ShapeDtypeStruct((B, DB), jnp.float32),
            jax.ShapeDtypeStruct((B * DG,), jnp.float32),
            jax.ShapeDtypeStruct((B * DT,), jnp.float32),
        ],
        scratch_types=[
            pltpu.VMEM((BPW,), jnp.int32),
            pltpu.VMEM((BPW * DG + 16,), jnp.int32),
            pltpu.VMEM((1, DB), jnp.float32),
            pltpu.VMEM((BPW, DB), jnp.float32),
            pltpu.VMEM((BPW * DG,), jnp.float32),
            pltpu.VMEM((BPW * DT,), jnp.float32),
            pltpu.SemaphoreType.DMA,
            pltpu.SemaphoreType.DMA,
        ],
    )(_body_narrow)

    run_wide = functools.partial(
        pl.kernel,
        mesh=mesh,
        compiler_params=cp,
        out_type=[jax.ShapeDtypeStruct((B * DP,), jnp.float32)],
        scratch_types=[
            pltpu.VMEM((BPW,), jnp.int32),
            pltpu.VMEM((BPW * DP + 16,), jnp.int32),
            pltpu.VMEM((BPW * DP,), jnp.float32),
            pltpu.SemaphoreType.DMA,
            pltpu.SemaphoreType.DMA,
        ],
    )(_body_wide)

    ob, ogo, otr = run_narrow(idx, betas, go_f, tr_f)
    (obp,) = run_wide(idx, bp_f)
    return (ob, ogo.reshape(B, DG), obp.reshape(B, DP), otr.reshape(B, DT))


# speedup vs baseline: 1.4545x; 1.0046x over previous
import functools

import jax
import jax.numpy as jnp
from jax import lax
from jax.experimental import pallas as pl
from jax.experimental.pallas import tpu as pltpu
from jax.experimental.pallas import tpu_sc as plsc

B = 4096
NC = 2
NS = 16
NW = NC * NS
BPW = B // NW
L = 16
DB = 10
DG = 3
DP = 69
DT = 3
N = 100000


def _worker_base():
    wid = lax.axis_index("s") * NC + lax.axis_index("c")
    return wid * BPW


def _body_narrow(idx_hbm, betas_hbm, go_hbm, tr_hbm,
                 out_b, out_go, out_tr,
                 idx_v, ego, bet_v, b_rows, go_rows, tr_rows,
                 sem, osem):
    base = _worker_base()
    pltpu.sync_copy(idx_hbm.at[pl.ds(base, BPW)], idx_v)
    pltpu.sync_copy(betas_hbm, bet_v)

    iota = lax.iota(jnp.int32, L)
    zeros = jnp.zeros((L,), jnp.int32)
    jconst0 = iota * N

    def row(b, _):
        ivec = plsc.load_gather(idx_v, [jnp.full((L,), b, jnp.int32)])
        plsc.store_scatter(ego, [DG * b + iota], jconst0 + ivec)
        return _

    lax.fori_loop(0, BPW, row, None)

    cps = [
        pltpu.async_copy(go_hbm.at[ego.at[pl.ds(0, BPW * DG)]], go_rows, sem),
        pltpu.async_copy(tr_hbm.at[ego.at[pl.ds(0, BPW * DG)]], tr_rows, sem),
    ]

    for m in range(5):
        lane = iota + 16 * m
        col = lane
        rsub = zeros
        for t in (10, 20, 30, 40, 50, 60, 70):
            col = jnp.where(lane >= t, lane - t, col)
            rsub = rsub + jnp.where(lane >= t, 1, 0)
        vm = plsc.load_gather(bet_v, [zeros, col])
        for r in range(BPW * DB // 80):
            plsc.store_scatter(b_rows, [8 * r + rsub, col], vm)

    for cp in cps:
        cp.wait()

    ocps = [
        pltpu.async_copy(go_rows, out_go.at[pl.ds(base * DG, BPW * DG)], osem),
        pltpu.async_copy(tr_rows, out_tr.at[pl.ds(base * DT, BPW * DT)], osem),
        pltpu.async_copy(b_rows, out_b.at[pl.ds(base, BPW)], osem),
    ]
    for cp in ocps:
        cp.wait()


def _body_wide(idx_hbm, bp_hbm, out_bp, idx_v, ebp, bp_rows, sem, osem):
    base = _worker_base()
    pltpu.sync_copy(idx_hbm.at[pl.ds(base, BPW)], idx_v)

    iota = lax.iota(jnp.int32, L)
    jconst = [(16 * k + iota) * N for k in range(5)]

    def row(b, _):
        ivec = plsc.load_gather(idx_v, [jnp.full((L,), b, jnp.int32)])
        for k in range(5):
            plsc.store_scatter(ebp, [DP * b + 16 * k + iota],
                               jconst[k] + ivec)
        return _

    NGRP = 4
    RG = BPW // NGRP
    EG = RG * DP
    cps = []
    for g in range(NGRP):
        lax.fori_loop(g * RG, (g + 1) * RG, row, None)
        cps.append(pltpu.async_copy(
            bp_hbm.at[ebp.at[pl.ds(g * EG, EG)]],
            bp_rows.at[pl.ds(g * EG, EG)], sem))
    for cp in cps:
        cp.wait()
    pltpu.async_copy(bp_rows, out_bp.at[pl.ds(base * DP, BPW * DP)],
                     osem).wait()


def kernel(idx, betas, global_orient, body_pose, transl):
    idx = idx.astype(jnp.int32)
    go_f = global_orient.T.reshape(-1)
    bp_f = body_pose.T.reshape(-1)
    tr_f = transl.T.reshape(-1)
    mesh = plsc.VectorSubcoreMesh(core_axis_name="c", subcore_axis_name="s")
    cp = pltpu.CompilerParams(needs_layout_passes=False)

    run_narrow = functools.partial(
        pl.kernel,
        mesh=mesh,
        compiler_params=cp,
        out_type=[
            jax.ShapeDtypeStruct((B, DB), jnp.float32),
            jax.ShapeDtypeStruct((B * DG,), jnp.float32),
            jax.ShapeDtypeStruct((B * DT,), jnp.float32),
        ],
        scratch_types=[
            pltpu.VMEM((BPW,), jnp.int32),
            pltpu.VMEM((BPW * DG + 16,), jnp.int32),
            pltpu.VMEM((1, DB), jnp.float32),
            pltpu.VMEM((BPW, DB), jnp.float32),
            pltpu.VMEM((BPW * DG,), jnp.float32),
            pltpu.VMEM((BPW * DT,), jnp.float32),
            pltpu.SemaphoreType.DMA,
            pltpu.SemaphoreType.DMA,
        ],
    )(_body_narrow)

    run_wide = functools.partial(
        pl.kernel,
        mesh=mesh,
        compiler_params=cp,
        out_type=[jax.ShapeDtypeStruct((B * DP,), jnp.float32)],
        scratch_types=[
            pltpu.VMEM((BPW,), jnp.int32),
            pltpu.VMEM((BPW * DP + 16,), jnp.int32),
            pltpu.VMEM((BPW * DP,), jnp.float32),
            pltpu.SemaphoreType.DMA,
            pltpu.SemaphoreType.DMA,
        ],
    )(_body_wide)

    ob, ogo, otr = run_narrow(idx, betas, go_f, tr_f)
    (obp,) = run_wide(idx, bp_f)
    return (ob, ogo.reshape(B, DG), obp.reshape(B, DP), otr.reshape(B, DT))
